# Initial kernel scaffold; baseline (speedup 1.0000x reference)
import jax
import jax.numpy as jnp
from jax import lax
from jax.experimental import pallas as pl
from jax.experimental.pallas import tpu as pltpu
from jax.experimental.pallas import tpu_sc as plsc

N = 10000
E = 160000
D = 256
P = 5
NP = N * P
L = 16
NC, NS = 2, 16
NW = NC * NS

PW = 1568
PW_TAIL = NP - 31 * PW
FB = 112
ORG_ROWS = 312
ORG_CH = 104
EW = 4992

_STEPS = (8192, 4096, 2048, 1024, 512, 256, 128, 64, 32, 16, 8, 4, 2, 1)


def _body(orgf, orge, pm, genf, outf, oute,
          pmv, goffs, sidx, esrc, edst, estg, zbuf, gbuf, sem):
    c = lax.axis_index("c")
    s = lax.axis_index("s")
    w = c * NS + s
    iota = lax.iota(jnp.int32, L)
    zrow16 = jnp.zeros((L,), jnp.float32)

    r0 = w * ORG_ROWS
    for h in range(0, ORG_ROWS, ORG_CH):
        pltpu.sync_copy(orgf.at[pl.ds(r0 + h, ORG_CH)], gbuf.at[pl.ds(0, ORG_CH)])
        pltpu.sync_copy(gbuf.at[pl.ds(0, ORG_CH)], outf.at[pl.ds(r0 + h, ORG_CH)])

    @pl.when(w == 0)
    def _():
        pltpu.sync_copy(orgf.at[pl.ds(NW * ORG_ROWS, 16)], gbuf.at[pl.ds(0, 16)])
        pltpu.sync_copy(gbuf.at[pl.ds(0, 16)], outf.at[pl.ds(NW * ORG_ROWS, 16)])

    e0 = w * EW
    for r in (0, 1):
        pltpu.sync_copy(orge.at[r, pl.ds(e0, EW)], estg)
        pltpu.sync_copy(estg, oute.at[r, pl.ds(e0, EW)])

    @pl.when(w == 0)
    def _():
        for r in (0, 1):
            pltpu.sync_copy(orge.at[r, pl.ds(NW * EW, 256)], estg.at[pl.ds(0, 256)])
            pltpu.sync_copy(estg.at[pl.ds(0, 256)], oute.at[r, pl.ds(NW * EW, 256)])

    pltpu.sync_copy(pm, pmv)

    def pass_a(v, carry):
        x = pmv[pl.ds(v * L, L)]
        t = x.astype(jnp.int32)
        f = x - t.astype(jnp.float32)
        up = (f > 0.5) | ((f == 0.5) & (lax.rem(t, 2) != 0))
        k = jnp.clip(t + up.astype(jnp.int32), 0, P)
        inc = plsc.cumsum(k)
        goffs[pl.ds(v * L, L)] = (inc - k) + carry
        return carry + jnp.sum(k)

    S = lax.fori_loop(0, N // L, pass_a, jnp.int32(0))

    def zero_rows(ref, lo, hi):
        def zr(r, _):
            row = ref.at[r]
            for cc in range(D // L):
                row[pl.ds(cc * L, L)] = zrow16
            return 0
        lax.fori_loop(lo, hi, zr, 0)

    zero_rows(zbuf, 0, FB)

    def tile_work(pbase, npw):
        nv = npw // L

        def search(v, _):
            pvec = pbase + v * L + iota
            c0 = jnp.zeros((L,), jnp.int32)
            for step in _STEPS:
                idx = c0 + (step - 1)
                g = plsc.load_gather(goffs, [jnp.minimum(idx, N - 1)])
                ok = (idx < N) & (g <= pvec)
                c0 = jnp.where(ok, c0 + step, c0)
            i = c0 - 1
            gi = plsc.load_gather(goffs, [i])
            j = pvec - gi
            valid = pvec < S
            sidx[pl.ds(v * L, L)] = jnp.where(valid, i * P + j, 0)
            ival = jnp.where(valid, i, -1)
            nval = jnp.where(valid, N + pvec, -1)
            ev = v * (2 * L) + iota * 2
            plsc.store_scatter(esrc, [ev], ival)
            plsc.store_scatter(esrc, [ev + 1], nval)
            plsc.store_scatter(edst, [ev], nval)
            plsc.store_scatter(edst, [ev + 1], ival)
            return 0

        lax.fori_loop(0, nv, search, 0)

        pltpu.sync_copy(esrc.at[pl.ds(0, 2 * npw)],
                        oute.at[0, pl.ds(E + 2 * pbase, 2 * npw)])
        pltpu.sync_copy(edst.at[pl.ds(0, 2 * npw)],
                        oute.at[1, pl.ds(E + 2 * pbase, 2 * npw)])

        def do_batch(pstart, boff, bs):
            vb = jnp.clip(S - pstart, 0, bs)

            def gpath():
                cp = pltpu.async_copy(genf.at[sidx.at[pl.ds(boff, bs)]],
                                      gbuf.at[pl.ds(0, bs)], sem)
                cp.wait()
                zero_rows(gbuf, vb, bs)
                pltpu.sync_copy(gbuf.at[pl.ds(0, bs)],
                                outf.at[pl.ds(N + pstart, bs)])

            def zpath():
                pltpu.sync_copy(zbuf.at[pl.ds(0, bs)],
                                outf.at[pl.ds(N + pstart, bs)])

            lax.cond(vb > 0, gpath, zpath)

        nb = npw // FB

        def batch(b, _):
            do_batch(pbase + b * FB, b * FB, FB)
            return 0

        lax.fori_loop(0, nb, batch, 0)
        rem = npw - nb * FB
        if rem:
            do_batch(pbase + nb * FB, nb * FB, rem)

    lax.cond(w < NW - 1,
             lambda: tile_work(w * PW, PW),
             lambda: tile_work((NW - 1) * PW, PW_TAIL))


_mesh = plsc.VectorSubcoreMesh(core_axis_name="c", subcore_axis_name="s",
                               num_cores=NC, num_subcores=NS)

_sc_call = pl.kernel(
    _body,
    out_type=[
        jax.ShapeDtypeStruct((N + NP, D), jnp.float32),
        jax.ShapeDtypeStruct((2, E + 2 * NP), jnp.int32),
    ],
    mesh=_mesh,
    scratch_types=[
        pltpu.VMEM((N,), jnp.float32),
        pltpu.VMEM((N,), jnp.int32),
        pltpu.VMEM((PW,), jnp.int32),
        pltpu.VMEM((2 * PW,), jnp.int32),
        pltpu.VMEM((2 * PW,), jnp.int32),
        pltpu.VMEM((EW,), jnp.int32),
        pltpu.VMEM((FB, D), jnp.float32),
        pltpu.VMEM((FB, D), jnp.float32),
        pltpu.SemaphoreType.DMA,
    ],
)


@jax.jit
def kernel(org_feats, org_edges, pred_missing, gen_feats):
    genflat = gen_feats.reshape(NP, D)
    fill_feats, fill_edges = _sc_call(org_feats, org_edges.astype(jnp.int32),
                                      pred_missing, genflat)
    return fill_feats, fill_edges


# speedup vs baseline: 6.2586x; 6.2586x over previous
import jax
import jax.numpy as jnp
from jax import lax
from jax.experimental import pallas as pl
from jax.experimental.pallas import tpu as pltpu
from jax.experimental.pallas import tpu_sc as plsc

N = 10000
E = 160000
D = 256
P = 5
NP = N * P
L = 16
NC, NS = 2, 16
NW = NC * NS

PW = 1568
PW_TAIL = NP - 31 * PW
FB = 112
ORG_ROWS = 312
ORG_CH = 104
EW = 4992

_STEPS = (8192, 4096, 2048, 1024, 512, 256, 128, 64, 32, 16, 8, 4, 2, 1)


def _body(orgf, orge, pm, genf, outf, oute,
          pmv, goffs, sidx, esrc, edst, estg, zbuf, gbuf, sem):
    c = lax.axis_index("c")
    s = lax.axis_index("s")
    w = c * NS + s
    iota = lax.iota(jnp.int32, L)
    zrow16 = jnp.zeros((L,), jnp.float32)

    r0 = w * ORG_ROWS
    for h in range(0, ORG_ROWS, ORG_CH):
        pltpu.sync_copy(orgf.at[pl.ds(r0 + h, ORG_CH)], gbuf.at[pl.ds(0, ORG_CH)])
        pltpu.sync_copy(gbuf.at[pl.ds(0, ORG_CH)], outf.at[pl.ds(r0 + h, ORG_CH)])

    @pl.when(w == 0)
    def _():
        pltpu.sync_copy(orgf.at[pl.ds(NW * ORG_ROWS, 16)], gbuf.at[pl.ds(0, 16)])
        pltpu.sync_copy(gbuf.at[pl.ds(0, 16)], outf.at[pl.ds(NW * ORG_ROWS, 16)])

    e0 = w * EW
    for r in (0, 1):
        pltpu.sync_copy(orge.at[r, pl.ds(e0, EW)], estg)
        pltpu.sync_copy(estg, oute.at[r, pl.ds(e0, EW)])

    @pl.when(w == 0)
    def _():
        for r in (0, 1):
            pltpu.sync_copy(orge.at[r, pl.ds(NW * EW, 256)], estg.at[pl.ds(0, 256)])
            pltpu.sync_copy(estg.at[pl.ds(0, 256)], oute.at[r, pl.ds(NW * EW, 256)])

    pltpu.sync_copy(pm, pmv)

    def pass_a(v, carry):
        x = pmv[pl.ds(v * L, L)]
        t = x.astype(jnp.int32)
        f = x - t.astype(jnp.float32)
        up = (f > 0.5) | ((f == 0.5) & (lax.rem(t, 2) != 0))
        k = jnp.clip(t + jnp.where(up, jnp.int32(1), jnp.int32(0)), 0, P)
        inc = plsc.cumsum(k)
        goffs[pl.ds(v * L, L)] = (inc - k) + carry
        return carry + jnp.sum(k)

    S = lax.fori_loop(0, N // L, pass_a, jnp.int32(0))

    def zero_rows(ref, lo, hi):
        def zr(r, _):
            row = ref.at[r]
            for cc in range(D // L):
                row[pl.ds(cc * L, L)] = zrow16
            return 0
        lax.fori_loop(lo, hi, zr, 0)

    zero_rows(zbuf, 0, FB)

    def tile_work(pbase, npw):
        nv = npw // L

        def search(v, _):
            pvec = pbase + v * L + iota
            c0 = jnp.zeros((L,), jnp.int32)
            for step in _STEPS:
                idx = c0 + (step - 1)
                g = plsc.load_gather(goffs, [jnp.minimum(idx, N - 1)])
                ok = (idx < N) & (g <= pvec)
                c0 = jnp.where(ok, c0 + step, c0)
            i = c0 - 1
            gi = plsc.load_gather(goffs, [i])
            j = pvec - gi
            valid = pvec < S
            sidx[pl.ds(v * L, L)] = jnp.where(valid, i * P + j, 0)
            ival = jnp.where(valid, i, -1)
            nval = jnp.where(valid, N + pvec, -1)
            ev = v * (2 * L) + iota * 2
            plsc.store_scatter(esrc, [ev], ival)
            plsc.store_scatter(esrc, [ev + 1], nval)
            plsc.store_scatter(edst, [ev], nval)
            plsc.store_scatter(edst, [ev + 1], ival)
            return 0

        lax.fori_loop(0, nv, search, 0)

        pltpu.sync_copy(esrc.at[pl.ds(0, 2 * npw)],
                        oute.at[0, pl.ds(E + 2 * pbase, 2 * npw)])
        pltpu.sync_copy(edst.at[pl.ds(0, 2 * npw)],
                        oute.at[1, pl.ds(E + 2 * pbase, 2 * npw)])

        def do_batch(pstart, boff, bs):
            vb = jnp.clip(S - pstart, 0, bs)

            def gpath():
                cp = pltpu.async_copy(genf.at[sidx.at[pl.ds(boff, bs)]],
                                      gbuf.at[pl.ds(0, bs)], sem)
                cp.wait()
                zero_rows(gbuf, vb, bs)
                pltpu.sync_copy(gbuf.at[pl.ds(0, bs)],
                                outf.at[pl.ds(N + pstart, bs)])

            def zpath():
                pltpu.sync_copy(zbuf.at[pl.ds(0, bs)],
                                outf.at[pl.ds(N + pstart, bs)])

            lax.cond(vb > 0, gpath, zpath)

        nb = npw // FB

        def batch(b, _):
            do_batch(pbase + b * FB, b * FB, FB)
            return 0

        lax.fori_loop(0, nb, batch, 0)
        rem = npw - nb * FB
        if rem:
            do_batch(pbase + nb * FB, nb * FB, rem)

    lax.cond(w < NW - 1,
             lambda: tile_work(w * PW, PW),
             lambda: tile_work((NW - 1) * PW, PW_TAIL))


_mesh = plsc.VectorSubcoreMesh(core_axis_name="c", subcore_axis_name="s",
                               num_cores=NC, num_subcores=NS)

_sc_call = pl.kernel(
    _body,
    out_type=[
        jax.ShapeDtypeStruct((N + NP, D), jnp.float32),
        jax.ShapeDtypeStruct((2, E + 2 * NP), jnp.int32),
    ],
    mesh=_mesh,
    scratch_types=[
        pltpu.VMEM((N,), jnp.float32),
        pltpu.VMEM((N,), jnp.int32),
        pltpu.VMEM((PW,), jnp.int32),
        pltpu.VMEM((2 * PW,), jnp.int32),
        pltpu.VMEM((2 * PW,), jnp.int32),
        pltpu.VMEM((EW,), jnp.int32),
        pltpu.VMEM((FB, D), jnp.float32),
        pltpu.VMEM((FB, D), jnp.float32),
        pltpu.SemaphoreType.DMA,
    ],
    compiler_params=pltpu.CompilerParams(use_tc_tiling_on_sc=False,
                                         needs_layout_passes=False),
)


@jax.jit
def kernel(org_feats, org_edges, pred_missing, gen_feats):
    genflat = gen_feats.reshape(NP, D)
    fill_feats, fill_edges = _sc_call(org_feats, org_edges.astype(jnp.int32),
                                      pred_missing, genflat)
    return fill_feats, fill_edges
